# Initial kernel scaffold; baseline (speedup 1.0000x reference)
import jax
import jax.numpy as jnp
from jax.experimental import pallas as pl


def kernel(x, edge_index, node_prompt, anchor_prompt, w_weight, w_bias, layer):
    raise NotImplementedError("write your pallas kernel here")


# speedup vs baseline: 10.9672x; 10.9672x over previous
import functools

import jax
import jax.numpy as jnp
from jax import lax
from jax.experimental import pallas as pl
from jax.experimental.pallas import tpu as pltpu
from jax.experimental.pallas import tpu_sc as plsc

NC = 2
NS = 16
NW = NC * NS
LANES = 16


def _tc_prompt_proj(x_ref, w_ref, prompt_ref, bias_ref, outx_ref, pt_ref):
    xb = x_ref[...]
    outx_ref[...] = xb + prompt_ref[...]
    pt = lax.dot_general(
        w_ref[...], xb, (((1,), (1,)), ((), ())),
        preferred_element_type=jnp.float32,
    )
    pt_ref[...] = pt + bias_ref[...][:, 0:1]


def _tc_anchor_matmul(bt_ref, anc_ref, out_ref):
    out_ref[...] = lax.dot_general(
        bt_ref[...], anc_ref[...], (((0,), (0,)), ((), ())),
        preferred_element_type=jnp.float32,
    )


def _sc_edge_softmax(A, N, E, C, epw, pt_hbm, ei_hbm, out_hbm,
                     table, sidx, didx, obuf):
    cid = lax.axis_index("c")
    sid = lax.axis_index("s")
    wid = sid * NC + cid
    pltpu.sync_copy(pt_hbm, table)
    base0 = wid * epw

    def chunk_body(k, carry):
        base = base0 + k * C
        pltpu.sync_copy(ei_hbm.at[pl.ds(base, C)], sidx)
        pltpu.sync_copy(ei_hbm.at[pl.ds(E + base, C)], didx)

        def group_body(g, carry2):
            off = g * LANES
            si = sidx[pl.ds(off, LANES)]
            di = didx[pl.ds(off, LANES)]
            logits = []
            for a in range(A):
                ls = plsc.load_gather(table, [si + jnp.int32(a * N)])
                ld = plsc.load_gather(table, [di + jnp.int32((A + a) * N)])
                l = ls + ld
                logits.append(jnp.maximum(l, 0.01 * l))
            m = logits[0]
            for a in range(1, A):
                m = jnp.maximum(m, logits[a])
            exps = [jnp.exp(l - m) for l in logits]
            tot = exps[0]
            for a in range(1, A):
                tot = tot + exps[a]
            r = 1.0 / tot
            for a in range(A):
                obuf[pl.ds(a * C + off, LANES)] = exps[a] * r
            zero = jnp.zeros((LANES,), jnp.float32)
            for a in range(A, 8):
                obuf[pl.ds(a * C + off, LANES)] = zero
            return carry2

        lax.fori_loop(0, C // LANES, group_body, 0)
        for a in range(8):
            pltpu.sync_copy(obuf.at[pl.ds(a * C, C)],
                            out_hbm.at[pl.ds(a * E + base, C)])
        return carry

    lax.fori_loop(0, epw // C, chunk_body, 0)


def kernel(x, edge_index, node_prompt, anchor_prompt, w_weight, w_bias, layer):
    N, D = x.shape
    E = edge_index.shape[1]
    A = w_weight.shape[0]

    w_cat = jnp.concatenate([w_weight[:, :D], w_weight[:, D:]], axis=0)
    bias_cat = jnp.concatenate([w_bias, jnp.zeros((A,), jnp.float32)])
    bias_cat = jnp.broadcast_to(bias_cat[:, None], (2 * A, 128))

    outx, pt = pl.pallas_call(
        _tc_prompt_proj,
        out_shape=(
            jax.ShapeDtypeStruct((N, D), jnp.float32),
            jax.ShapeDtypeStruct((2 * A, N), jnp.float32),
        ),
    )(x, w_cat, node_prompt, bias_cat)

    epw = E // NW
    C = 2000
    mesh = plsc.VectorSubcoreMesh(core_axis_name="c", subcore_axis_name="s")
    sc_fn = pl.kernel(
        functools.partial(_sc_edge_softmax, A, N, E, C, epw),
        out_type=jax.ShapeDtypeStruct((8 * E,), jnp.float32),
        mesh=mesh,
        compiler_params=pltpu.CompilerParams(needs_layout_passes=False),
        scratch_types=[
            pltpu.VMEM((2 * A * N,), jnp.float32),
            pltpu.VMEM((C,), jnp.int32),
            pltpu.VMEM((C,), jnp.int32),
            pltpu.VMEM((8 * C,), jnp.float32),
        ],
    )
    bt = sc_fn(pt.reshape(2 * A * N), edge_index.reshape(2 * E)).reshape(8, E)

    anchor_pad = jnp.concatenate(
        [anchor_prompt, jnp.zeros((8 - A, D), jnp.float32)], axis=0)

    EB = 6400
    edge_prompt = pl.pallas_call(
        _tc_anchor_matmul,
        grid=(E // EB,),
        in_specs=[
            pl.BlockSpec((8, EB), lambda i: (0, i)),
            pl.BlockSpec((8, D), lambda i: (0, 0)),
        ],
        out_specs=pl.BlockSpec((EB, D), lambda i: (i, 0)),
        out_shape=jax.ShapeDtypeStruct((E, D), jnp.float32),
    )(bt, anchor_pad)

    return (outx, edge_prompt)


# speedup vs baseline: 12.3460x; 1.1257x over previous
import functools

import jax
import jax.numpy as jnp
from jax import lax
from jax.experimental import pallas as pl
from jax.experimental.pallas import tpu as pltpu
from jax.experimental.pallas import tpu_sc as plsc

NC = 2
NS = 16
NW = NC * NS
LANES = 16


def _tc_prompt_proj(x_ref, w_ref, prompt_ref, bias_ref, outx_ref, pt_ref):
    xb = x_ref[...]
    outx_ref[...] = xb + prompt_ref[...]
    pt = lax.dot_general(
        w_ref[...], xb, (((1,), (1,)), ((), ())),
        preferred_element_type=jnp.float32,
    )
    pt_ref[...] = pt + bias_ref[...][:, 0:1]


def _tc_anchor_matmul(bt_ref, anc_ref, out_ref):
    out_ref[...] = lax.dot_general(
        bt_ref[...], anc_ref[...], (((0,), (0,)), ((), ())),
        preferred_element_type=jnp.float32,
    )


def _sc_edge_softmax(A, N, E, C, epw, pt_hbm, ei_hbm, out_hbm,
                     table, sidx, didx, obuf):
    cid = lax.axis_index("c")
    sid = lax.axis_index("s")
    wid = sid * NC + cid
    pltpu.sync_copy(pt_hbm, table)
    base0 = wid * epw

    def chunk_body(k, carry):
        base = base0 + k * C
        pltpu.sync_copy(ei_hbm.at[pl.ds(base, C)], sidx)
        pltpu.sync_copy(ei_hbm.at[pl.ds(E + base, C)], didx)

        def group_body(g, carry2):
            off = g * LANES
            si = sidx[pl.ds(off, LANES)]
            di = didx[pl.ds(off, LANES)]
            logits = []
            for a in range(A):
                ls = plsc.load_gather(table, [si + jnp.int32(a * N)])
                ld = plsc.load_gather(table, [di + jnp.int32((A + a) * N)])
                l = ls + ld
                logits.append(jnp.maximum(l, 0.01 * l))
            m = logits[0]
            for a in range(1, A):
                m = jnp.maximum(m, logits[a])
            exps = [jnp.exp(l - m) for l in logits]
            tot = exps[0]
            for a in range(1, A):
                tot = tot + exps[a]
            r = 1.0 / tot
            for a in range(A):
                obuf[pl.ds(a * C + off, LANES)] = exps[a] * r
            return carry2

        lax.fori_loop(0, C // LANES, group_body, 0)
        for a in range(A):
            pltpu.sync_copy(obuf.at[pl.ds(a * C, C)],
                            out_hbm.at[pl.ds(a * E + base, C)])
        return carry

    lax.fori_loop(0, epw // C, chunk_body, 0)


def kernel(x, edge_index, node_prompt, anchor_prompt, w_weight, w_bias, layer):
    N, D = x.shape
    E = edge_index.shape[1]
    A = w_weight.shape[0]

    w_cat = jnp.concatenate([w_weight[:, :D], w_weight[:, D:]], axis=0)
    bias_cat = jnp.concatenate([w_bias, jnp.zeros((A,), jnp.float32)])
    bias_cat = jnp.broadcast_to(bias_cat[:, None], (2 * A, 128))

    outx, pt = pl.pallas_call(
        _tc_prompt_proj,
        out_shape=(
            jax.ShapeDtypeStruct((N, D), jnp.float32),
            jax.ShapeDtypeStruct((2 * A, N), jnp.float32),
        ),
    )(x, w_cat, node_prompt, bias_cat)

    epw = E // NW
    C = 2000
    mesh = plsc.VectorSubcoreMesh(core_axis_name="c", subcore_axis_name="s")
    sc_fn = pl.kernel(
        functools.partial(_sc_edge_softmax, A, N, E, C, epw),
        out_type=jax.ShapeDtypeStruct((A * E,), jnp.float32),
        mesh=mesh,
        compiler_params=pltpu.CompilerParams(needs_layout_passes=False),
        scratch_types=[
            pltpu.VMEM((2 * A * N,), jnp.float32),
            pltpu.VMEM((C,), jnp.int32),
            pltpu.VMEM((C,), jnp.int32),
            pltpu.VMEM((A * C,), jnp.float32),
        ],
    )
    bt = sc_fn(pt.reshape(2 * A * N), edge_index.reshape(2 * E)).reshape(A, E)

    EB = 12800
    edge_prompt = pl.pallas_call(
        _tc_anchor_matmul,
        grid=(E // EB,),
        in_specs=[
            pl.BlockSpec((A, EB), lambda i: (0, i)),
            pl.BlockSpec((A, D), lambda i: (0, 0)),
        ],
        out_specs=pl.BlockSpec((EB, D), lambda i: (i, 0)),
        out_shape=jax.ShapeDtypeStruct((E, D), jnp.float32),
    )(bt, anchor_prompt)

    return (outx, edge_prompt)
